# baseline (speedup 1.0000x reference)
import functools

import jax
import jax.numpy as jnp
from jax import lax
from jax.experimental import pallas as pl
from jax.experimental.pallas import tpu as pltpu
from jax.experimental.pallas import tpu_sc as plsc

_VOCAB = 100000
_EMBED = 64
_BATCH = 4096
_CTX = 20

_NC = 2
_NS = 16
_NW = _NC * _NS
_IDX_TOTAL = _BATCH * _CTX
_PER_W = _IDX_TOTAL // _NW
_CHUNK = 512
_NCHUNK = _PER_W // _CHUNK
_PAIR = 2 * _EMBED


def _sc_gather(table_pairs, idx_flat):
    mesh = plsc.VectorSubcoreMesh(core_axis_name="c", subcore_axis_name="s")

    @functools.partial(
        pl.kernel,
        mesh=mesh,
        out_type=jax.ShapeDtypeStruct((_IDX_TOTAL, _PAIR), jnp.float32),
        scratch_types=[
            pltpu.VMEM((_CHUNK,), jnp.int32),
            pltpu.VMEM((_CHUNK, _PAIR), jnp.float32),
            pltpu.SemaphoreType.DMA,
        ],
    )
    def k(table_hbm, idx_hbm, out_hbm, idx_v, rows_v, sem):
        wid = lax.axis_index("s") * _NC + lax.axis_index("c")
        for c in range(_NCHUNK):
            base = wid * _PER_W + c * _CHUNK
            pltpu.sync_copy(idx_hbm.at[pl.ds(base, _CHUNK)], idx_v)
            pltpu.async_copy(table_hbm.at[idx_v], rows_v, sem).wait()
            pltpu.sync_copy(rows_v, out_hbm.at[pl.ds(base, _CHUNK)])

    return k(table_pairs, idx_flat)


_BM_MEAN = 512


def _mean_body(g_ref, p_ref, o_ref):
    g = g_ref[...]
    pf = p_ref[...][:, :, None]
    b = jnp.sum(g * pf, axis=0)
    s = jnp.sum(g, axis=0) - b
    o_ref[...] = (s[:, :_EMBED] + b[:, _EMBED:]) * (1.0 / _CTX)


def _mean_pool(g3, parity):
    return pl.pallas_call(
        _mean_body,
        grid=(_BATCH // _BM_MEAN,),
        in_specs=[
            pl.BlockSpec((_CTX, _BM_MEAN, _PAIR), lambda i: (0, i, 0)),
            pl.BlockSpec((_CTX, _BM_MEAN), lambda i: (0, i)),
        ],
        out_specs=pl.BlockSpec((_BM_MEAN, _EMBED), lambda i: (i, 0)),
        out_shape=jax.ShapeDtypeStruct((_BATCH, _EMBED), jnp.float32),
    )(g3, parity)


_BN = 1024


def _proj_body(x_ref, w_ref, b_ref, o_ref):
    o_ref[...] = lax.dot_general(
        x_ref[...], w_ref[...],
        (((1,), (1,)), ((), ())),
        preferred_element_type=jnp.float32,
    ) + b_ref[...]


def _project(x, lin_w, bias2d):
    nj = pl.cdiv(_VOCAB, _BN)
    return pl.pallas_call(
        _proj_body,
        grid=(nj,),
        in_specs=[
            pl.BlockSpec((_BATCH, _EMBED), lambda j: (0, 0)),
            pl.BlockSpec((_BN, _EMBED), lambda j: (j, 0)),
            pl.BlockSpec((1, _BN), lambda j: (0, j)),
        ],
        out_specs=pl.BlockSpec((_BATCH, _BN), lambda j: (0, j)),
        out_shape=jax.ShapeDtypeStruct((_BATCH, _VOCAB), jnp.float32),
    )(x, lin_w, bias2d)


def kernel(context_indices, emb_table, lin_w, lin_b):
    idx_cm = context_indices.astype(jnp.int32).T
    idx_pair = (idx_cm >> 1).reshape(_IDX_TOTAL)
    parity = (idx_cm & 1).astype(jnp.float32)
    table_pairs = emb_table.reshape(_VOCAB // 2, _PAIR)
    g = _sc_gather(table_pairs, idx_pair)
    g3 = g.reshape(_CTX, _BATCH, _PAIR)
    x = _mean_pool(g3, parity)
    return _project(x, lin_w, lin_b.reshape(1, _VOCAB))
